# Initial kernel scaffold; baseline (speedup 1.0000x reference)
import jax
import jax.numpy as jnp
from jax.experimental import pallas as pl


def kernel(x, c, ada_w, ada_b, router_w, out_w, out_b, exp_fc1_w, exp_fc1_b, exp_fc2_w, exp_fc2_b, mlp_fc1_w, mlp_fc1_b, mlp_fc2_w, mlp_fc2_b):
    raise NotImplementedError("write your pallas kernel here")


# speedup vs baseline: 3.5178x; 3.5178x over previous
import functools

import jax
import jax.numpy as jnp
from jax.experimental import pallas as pl
from jax.experimental.pallas import tpu as pltpu

B, N, D = 4, 1024, 768
HID = 3072
E = 10
MIX_HID = 1024


def _gelu(v):
    return jax.nn.gelu(v, approximate=True)


def _router_body(x_ref, c_ref, ada_w_ref, ada_b_ref, router_w_ref,
                 ada_ref, idx_ref, aux_ref):
    xm = jnp.mean(x_ref[...], axis=1)
    cc = c_ref[...]
    sc = cc * jax.nn.sigmoid(cc)
    ada = jax.lax.dot_general(sc, ada_w_ref[...], (((1,), (1,)), ((), ())),
                              preferred_element_type=jnp.float32)
    ada = ada + ada_b_ref[...]
    ada_ref[...] = ada
    shift_mom = ada[:, 0:D]
    scale_mom = ada[:, D:2 * D]
    ri = xm * (1.0 + scale_mom) + shift_mom
    logits = jax.lax.dot_general(ri, router_w_ref[...], (((1,), (1,)), ((), ())),
                                 preferred_element_type=jnp.float32)
    m = jnp.max(logits, axis=1, keepdims=True)
    p = jnp.exp(logits - m)
    probs = p / jnp.sum(p, axis=1, keepdims=True)
    iota_e = jax.lax.broadcasted_iota(jnp.int32, (B, E), 1)
    cand = jnp.where(logits == m, iota_e, E)
    idx = jnp.min(cand, axis=1)
    idx_ref[...] = jnp.broadcast_to(idx[:, None], (B, 128))
    onehot = (iota_e == idx[:, None]).astype(jnp.float32)
    aux = E * jnp.sum(jnp.mean(probs, axis=0) * jnp.mean(onehot, axis=0))
    aux_ref[...] = jnp.reshape(aux, (1, 1))


def _expert_body(idx_ref, x_ref, ada_ref, w1_ref, b1_ref, w2_ref, b2_ref,
                 out_ref):
    del idx_ref
    xb = x_ref[0]
    a = ada_ref[0]
    shift = a[:, 0:D]
    scale = a[:, D:2 * D]
    mx = xb * (1.0 + scale) + shift
    mu = jnp.mean(mx, axis=0, keepdims=True)
    var = jnp.mean((mx - mu) ** 2, axis=0, keepdims=True)
    xn = (mx - mu) * jax.lax.rsqrt(var + 1e-5)
    w1 = w1_ref[0]
    h = jax.lax.dot_general(xn, w1, (((0,), (1,)), ((), ())),
                            preferred_element_type=jnp.float32)
    g = _gelu(h + b1_ref[0])
    w2 = w2_ref[0]
    ot = jax.lax.dot_general(g, w2, (((1,), (1,)), ((), ())),
                             preferred_element_type=jnp.float32)
    out_ref[0] = ot + b2_ref[0]


def _post_body(eo_ref, x_ref, ada_ref, out_w_ref, out_b_ref,
               fc1_ref, fc1_b_ref, fc2_ref, fc2_b_ref, out_ref):
    eo = eo_ref[0]
    xb = x_ref[0]
    a = ada_ref[0]
    gate_mom = a[:, 2 * D:3 * D]
    shift_mlp = a[:, 3 * D:4 * D]
    scale_mlp = a[:, 4 * D:5 * D]
    gate_mlp = a[:, 5 * D:6 * D]
    y = jax.lax.dot_general(eo, out_w_ref[...], (((0,), (1,)), ((), ())),
                            preferred_element_type=jnp.float32)
    y = y + out_b_ref[...]
    x1 = xb + gate_mom * y
    mu = jnp.mean(x1, axis=1, keepdims=True)
    var = jnp.mean((x1 - mu) ** 2, axis=1, keepdims=True)
    xn2 = (x1 - mu) * jax.lax.rsqrt(var + 1e-6)
    mod = xn2 * (1.0 + scale_mlp) + shift_mlp
    h = jax.lax.dot_general(mod, fc1_ref[...], (((1,), (1,)), ((), ())),
                            preferred_element_type=jnp.float32)
    g = _gelu(h + fc1_b_ref[...])
    mlp = jax.lax.dot_general(g, fc2_ref[...], (((1,), (1,)), ((), ())),
                              preferred_element_type=jnp.float32)
    mlp = mlp + fc2_b_ref[...]
    out_ref[0] = x1 + gate_mlp * mlp


TN = 512


@jax.jit
def kernel(x, c, ada_w, ada_b, router_w, out_w, out_b,
           exp_fc1_w, exp_fc1_b, exp_fc2_w, exp_fc2_b,
           mlp_fc1_w, mlp_fc1_b, mlp_fc2_w, mlp_fc2_b):
    f32 = jnp.float32

    ada, idx_pad, aux = pl.pallas_call(
        _router_body,
        out_shape=(
            jax.ShapeDtypeStruct((B, 6 * D), f32),
            jax.ShapeDtypeStruct((B, 128), jnp.int32),
            jax.ShapeDtypeStruct((1, 1), f32),
        ),
    )(x, c, ada_w, ada_b.reshape(1, 6 * D), router_w)
    idx = idx_pad[:, 0]
    ada3 = ada.reshape(B, 1, 6 * D)

    grid_b = pltpu.PrefetchScalarGridSpec(
        num_scalar_prefetch=1,
        grid=(B,),
        in_specs=[
            pl.BlockSpec((1, N, D), lambda b, idx_ref: (b, 0, 0)),
            pl.BlockSpec((1, 1, 6 * D), lambda b, idx_ref: (b, 0, 0)),
            pl.BlockSpec((1, MIX_HID, N), lambda b, idx_ref: (idx_ref[b], 0, 0)),
            pl.BlockSpec((1, 1, MIX_HID), lambda b, idx_ref: (idx_ref[b], 0, 0)),
            pl.BlockSpec((1, N, MIX_HID), lambda b, idx_ref: (idx_ref[b], 0, 0)),
            pl.BlockSpec((1, 1, N), lambda b, idx_ref: (idx_ref[b], 0, 0)),
        ],
        out_specs=pl.BlockSpec((1, D, N), lambda b, idx_ref: (b, 0, 0)),
    )
    eo = pl.pallas_call(
        _expert_body,
        grid_spec=grid_b,
        out_shape=jax.ShapeDtypeStruct((B, D, N), f32),
    )(idx, x, ada3, exp_fc1_w, exp_fc1_b.reshape(E, 1, MIX_HID),
      exp_fc2_w, exp_fc2_b.reshape(E, 1, N))

    x2 = pl.pallas_call(
        _post_body,
        grid=(B, N // TN),
        in_specs=[
            pl.BlockSpec((1, D, TN), lambda b, t: (b, 0, t)),
            pl.BlockSpec((1, TN, D), lambda b, t: (b, t, 0)),
            pl.BlockSpec((1, 1, 6 * D), lambda b, t: (b, 0, 0)),
            pl.BlockSpec((D, D), lambda b, t: (0, 0)),
            pl.BlockSpec((1, D), lambda b, t: (0, 0)),
            pl.BlockSpec((HID, D), lambda b, t: (0, 0)),
            pl.BlockSpec((1, HID), lambda b, t: (0, 0)),
            pl.BlockSpec((D, HID), lambda b, t: (0, 0)),
            pl.BlockSpec((1, D), lambda b, t: (0, 0)),
        ],
        out_specs=pl.BlockSpec((1, TN, D), lambda b, t: (b, t, 0)),
        out_shape=jax.ShapeDtypeStruct((B, N, D), f32),
    )(eo, x, ada3, out_w, out_b.reshape(1, D),
      mlp_fc1_w, mlp_fc1_b.reshape(1, HID), mlp_fc2_w, mlp_fc2_b.reshape(1, D))

    return (x2, aux[0, 0])


# speedup vs baseline: 3.9544x; 1.1241x over previous
import functools

import jax
import jax.numpy as jnp
from jax.experimental import pallas as pl
from jax.experimental.pallas import tpu as pltpu

B, N, D = 4, 1024, 768
HID = 3072
E = 10
MIX_HID = 1024


def _gelu(v):
    return jax.nn.gelu(v, approximate=True)


def _router_body(x_ref, c_ref, ada_w_ref, ada_b_ref, router_w_ref,
                 ada_ref, idx_ref, aux_ref):
    xm = jnp.mean(x_ref[...], axis=1)
    cc = c_ref[...]
    sc = cc * jax.nn.sigmoid(cc)
    ada = jax.lax.dot_general(sc, ada_w_ref[...], (((1,), (1,)), ((), ())),
                              preferred_element_type=jnp.float32)
    ada = ada + ada_b_ref[...]
    ada_ref[...] = ada
    shift_mom = ada[:, 0:D]
    scale_mom = ada[:, D:2 * D]
    ri = xm * (1.0 + scale_mom) + shift_mom
    logits = jax.lax.dot_general(ri, router_w_ref[...], (((1,), (1,)), ((), ())),
                                 preferred_element_type=jnp.float32)
    m = jnp.max(logits, axis=1, keepdims=True)
    p = jnp.exp(logits - m)
    probs = p / jnp.sum(p, axis=1, keepdims=True)
    iota_e = jax.lax.broadcasted_iota(jnp.int32, (B, E), 1)
    cand = jnp.where(logits == m, iota_e, E)
    idx = jnp.min(cand, axis=1)
    idx_ref[...] = jnp.broadcast_to(idx[:, None], (B, 128))
    onehot = (iota_e == idx[:, None]).astype(jnp.float32)
    aux = E * jnp.sum(jnp.mean(probs, axis=0) * jnp.mean(onehot, axis=0))
    aux_ref[...] = jnp.reshape(aux, (1, 1))


def _block_body(idx_ref, x_ref, ada_ref, w1_ref, b1_ref, w2_ref, b2_ref,
                out_w_ref, out_b_ref, fc1_ref, fc1_b_ref, fc2_ref, fc2_b_ref,
                out_ref):
    del idx_ref
    xb = x_ref[0]
    a = ada_ref[0]
    shift = a[:, 0:D]
    scale = a[:, D:2 * D]
    gate_mom = a[:, 2 * D:3 * D]
    shift_mlp = a[:, 3 * D:4 * D]
    scale_mlp = a[:, 4 * D:5 * D]
    gate_mlp = a[:, 5 * D:6 * D]

    mx = xb * (1.0 + scale) + shift
    mu = jnp.mean(mx, axis=0, keepdims=True)
    var = jnp.mean((mx - mu) ** 2, axis=0, keepdims=True)
    xn = (mx - mu) * jax.lax.rsqrt(var + 1e-5)
    w1 = w1_ref[0]
    h = jax.lax.dot_general(xn, w1, (((0,), (1,)), ((), ())),
                            preferred_element_type=jnp.float32)
    g = _gelu(h + b1_ref[0])
    w2 = w2_ref[0]
    ot = jax.lax.dot_general(g, w2, (((1,), (1,)), ((), ())),
                             preferred_element_type=jnp.float32)
    ot = ot + b2_ref[0]

    y = jax.lax.dot_general(ot, out_w_ref[...], (((0,), (1,)), ((), ())),
                            preferred_element_type=jnp.float32)
    y = y + out_b_ref[...]
    TT = N // 2
    for t in range(2):
        x1 = xb[t * TT:(t + 1) * TT, :] + gate_mom * y[t * TT:(t + 1) * TT, :]
        mu2 = jnp.mean(x1, axis=1, keepdims=True)
        var2 = jnp.mean((x1 - mu2) ** 2, axis=1, keepdims=True)
        xn2 = (x1 - mu2) * jax.lax.rsqrt(var2 + 1e-6)
        mod = xn2 * (1.0 + scale_mlp) + shift_mlp
        hm = jax.lax.dot_general(mod, fc1_ref[...], (((1,), (1,)), ((), ())),
                                 preferred_element_type=jnp.float32)
        gm = _gelu(hm + fc1_b_ref[...])
        mlp = jax.lax.dot_general(gm, fc2_ref[...], (((1,), (1,)), ((), ())),
                                  preferred_element_type=jnp.float32)
        mlp = mlp + fc2_b_ref[...]
        out_ref[0, t * TT:(t + 1) * TT, :] = x1 + gate_mlp * mlp


@jax.jit
def kernel(x, c, ada_w, ada_b, router_w, out_w, out_b,
           exp_fc1_w, exp_fc1_b, exp_fc2_w, exp_fc2_b,
           mlp_fc1_w, mlp_fc1_b, mlp_fc2_w, mlp_fc2_b):
    f32 = jnp.float32

    ada, idx_pad, aux = pl.pallas_call(
        _router_body,
        out_shape=(
            jax.ShapeDtypeStruct((B, 6 * D), f32),
            jax.ShapeDtypeStruct((B, 128), jnp.int32),
            jax.ShapeDtypeStruct((1, 1), f32),
        ),
    )(x, c, ada_w, ada_b.reshape(1, 6 * D), router_w)
    idx = idx_pad[:, 0]
    ada3 = ada.reshape(B, 1, 6 * D)

    grid_b = pltpu.PrefetchScalarGridSpec(
        num_scalar_prefetch=1,
        grid=(B,),
        in_specs=[
            pl.BlockSpec((1, N, D), lambda b, idx_ref: (b, 0, 0)),
            pl.BlockSpec((1, 1, 6 * D), lambda b, idx_ref: (b, 0, 0)),
            pl.BlockSpec((1, MIX_HID, N), lambda b, idx_ref: (idx_ref[b], 0, 0)),
            pl.BlockSpec((1, 1, MIX_HID), lambda b, idx_ref: (idx_ref[b], 0, 0)),
            pl.BlockSpec((1, N, MIX_HID), lambda b, idx_ref: (idx_ref[b], 0, 0)),
            pl.BlockSpec((1, 1, N), lambda b, idx_ref: (idx_ref[b], 0, 0)),
            pl.BlockSpec((D, D), lambda b, idx_ref: (0, 0)),
            pl.BlockSpec((1, D), lambda b, idx_ref: (0, 0)),
            pl.BlockSpec((HID, D), lambda b, idx_ref: (0, 0)),
            pl.BlockSpec((1, HID), lambda b, idx_ref: (0, 0)),
            pl.BlockSpec((D, HID), lambda b, idx_ref: (0, 0)),
            pl.BlockSpec((1, D), lambda b, idx_ref: (0, 0)),
        ],
        out_specs=pl.BlockSpec((1, N, D), lambda b, idx_ref: (b, 0, 0)),
    )
    x2 = pl.pallas_call(
        _block_body,
        grid_spec=grid_b,
        out_shape=jax.ShapeDtypeStruct((B, N, D), f32),
        compiler_params=pltpu.CompilerParams(vmem_limit_bytes=100 * 1024 * 1024),
    )(idx, x, ada3, exp_fc1_w, exp_fc1_b.reshape(E, 1, MIX_HID),
      exp_fc2_w, exp_fc2_b.reshape(E, 1, N),
      out_w, out_b.reshape(1, D),
      mlp_fc1_w, mlp_fc1_b.reshape(1, HID), mlp_fc2_w, mlp_fc2_b.reshape(1, D))

    return (x2, aux[0, 0])
